# Initial kernel scaffold; baseline (speedup 1.0000x reference)
import jax
import jax.numpy as jnp
from jax.experimental import pallas as pl


def kernel(pred0, pred1, pred2, tbox0, tbox1, tbox2, anch0, anch1, anch2, b0, a0, gj0, gi0, tcls0, b1, a1, gj1, gi1, tcls1, b2, a2, gj2, gi2, tcls2):
    raise NotImplementedError("write your pallas kernel here")


# speedup vs baseline: 1.5070x; 1.5070x over previous
import functools
import math

import jax
import jax.numpy as jnp
from jax import lax
from jax.experimental import pallas as pl
from jax.experimental.pallas import tpu as pltpu
from jax.experimental.pallas import tpu_sc as plsc

_HW = (20, 40, 80)
_BS = 16
_NC = 80
_NM = 300
_NTILES = 32
_NPT = 10
_SLOT = 96
_TSLOTS = 1024
_BAL = (0.4, 1.0, 4.0)


def _sc_gather(p0f, p1f, p2f, idxs):
    mesh = plsc.VectorSubcoreMesh(core_axis_name="c", subcore_axis_name="s")
    f32 = jnp.float32
    i32 = jnp.int32

    @functools.partial(
        pl.kernel,
        mesh=mesh,
        out_type=[jax.ShapeDtypeStruct((_NTILES, _TSLOTS), f32) for _ in range(3)]
        + [jax.ShapeDtypeStruct((_NTILES, 16), f32) for _ in range(3)],
        scratch_types=[
            pltpu.VMEM((12 * _NM,), i32),
            pltpu.VMEM((_TSLOTS // 128, 128), i32),
            pltpu.VMEM((_TSLOTS,), f32),
            pltpu.VMEM((16,), f32),
            pltpu.SemaphoreType.DMA,
        ],
        compiler_params=pltpu.CompilerParams(needs_layout_passes=False,
                                             use_tc_tiling_on_sc=False),
    )
    def body(p0, p1, p2, idxs_hbm, out0, out1, out2, key0, key1, key2,
             idx_v, idxbuf, rows, keybuf, sem):
        cc = lax.axis_index("c")
        ss = lax.axis_index("s")
        wid = ss * 2 + cc
        pltpu.sync_copy(idxs_hbm, idx_v)
        iota = lax.iota(i32, 16)
        for lvl, (h, tbl, outr, keyr) in enumerate(
            ((20, p0, out0, key0), (40, p1, out1, key1), (80, p2, out2, key2))):
            hh = h * h
            o0 = 4 * lvl * _NM
            o1 = o0 + _NM
            o2 = o0 + 2 * _NM
            o3 = o0 + 3 * _NM
            n16 = jnp.minimum(wid * _NPT + iota, _NM - 1)
            bv = plsc.load_gather(idx_v, [n16 + o0])
            av = plsc.load_gather(idx_v, [n16 + o1])
            gjv = plsc.load_gather(idx_v, [n16 + o2])
            giv = plsc.load_gather(idx_v, [n16 + o3])
            keybuf[...] = (((bv * 3 + av) * h + gjv) * h + giv).astype(f32)
            pltpu.sync_copy(keybuf, keyr.at[wid])
            for nl in range(_NPT):
                nf = jnp.full((16,), jnp.minimum(wid * _NPT + nl, _NM - 1), i32)
                b1 = plsc.load_gather(idx_v, [nf + o0])
                a1 = plsc.load_gather(idx_v, [nf + o1])
                gj1 = plsc.load_gather(idx_v, [nf + o2])
                gi1 = plsc.load_gather(idx_v, [nf + o3])
                base = b1 * (255 * hh) + a1 * (85 * hh) + gj1 * h + gi1
                for k in range(6):
                    cvec = iota + 16 * k
                    gidx = base + cvec * hh
                    if 16 * (k + 1) > 85:
                        gidx = jnp.where(cvec < 85, gidx, 0)
                    slot = nl * _SLOT + k * 16
                    idxbuf[slot // 128, pl.ds(slot % 128, 16)] = gidx
            for j in range(4):
                slot = _NPT * _SLOT + j * 16
                idxbuf[slot // 128, pl.ds(slot % 128, 16)] = jnp.zeros((16,), i32)
            cps = [
                pltpu.async_copy(tbl.at[idxbuf.at[q]],
                                 rows.at[pl.ds(q * 128, 128)], sem)
                for q in range(_TSLOTS // 128)
            ]
            for cp in cps:
                cp.wait()
            pltpu.sync_copy(rows, outr.at[wid])

    return body(p0f, p1f, p2f, idxs)


def _softplus(x):
    return jnp.maximum(x, 0.0) + jnp.log(1.0 + jnp.exp(-jnp.abs(x)))


def _sigmoid(x):
    return 1.0 / (1.0 + jnp.exp(-x))


_ATAN_C = (9.999999990537e-01, -3.333329671515e-01, 1.999854226698e-01,
           -1.426438979378e-01, 1.095344985227e-01, -8.407879225914e-02,
           5.804045198803e-02, -3.126450654751e-02, 1.096244313837e-02,
           -1.804490179632e-03)


def _atan_pos(x):
    u = jnp.minimum(x, 1.0 / x)
    u2 = u * u
    p = jnp.float32(_ATAN_C[-1])
    for coef in _ATAN_C[-2::-1]:
        p = p * u2 + coef
    r = u * p
    return jnp.where(x > 1.0, jnp.float32(math.pi / 2) - r, r)


def _tc_loss_body(pr0, pr1, pr2,
                  pp0, pp1, pp2, kc0, kc1, kc2, kr0, kr1, kr2,
                  gi0, gj0, gi1, gj1, gi2, gj2,
                  tb0, tb1, tb2, an0, an1, an2, tc0, tc1, tc2,
                  out, acc):
    i = pl.program_id(0)
    j = pl.program_id(1)

    @pl.when(jnp.logical_and(i == 0, j == 0))
    def _init():
        for lvl in range(3):
            acc[lvl] = 0.0

    for lvl, pr in enumerate((pr0, pr1, pr2)):
        acc[lvl] = acc[lvl] + jnp.sum(_softplus(pr[...]))

    @pl.when(jnp.logical_and(i == _BS - 1, j == 2))
    def _final():
        eps = 1e-7
        box_loss = jnp.float32(0.0)
        cls_loss = jnp.float32(0.0)
        obj_loss = jnp.float32(0.0)
        ppl = (pp0, pp1, pp2)
        kcl = (kc0, kc1, kc2)
        krl = (kr0, kr1, kr2)
        gil = (gi0, gi1, gi2)
        gjl = (gj0, gj1, gj2)
        tbl = (tb0, tb1, tb2)
        anl = (an0, an1, an2)
        tcl = (tc0, tc1, tc2)
        for lvl, h in enumerate(_HW):
            pp = ppl[lvl][...]
            gi = gil[lvl][...]
            gj = gjl[lvl][...]
            tb = tbl[lvl][...]
            an = anl[lvl][...]
            hf = jnp.float32(h)
            px = _sigmoid(pp[:, 0:1]) * 2.0 - 0.5
            py = _sigmoid(pp[:, 1:2]) * 2.0 - 0.5
            pw = (_sigmoid(pp[:, 2:3]) * 2.0) ** 2 * an[:, 0:1]
            ph = (_sigmoid(pp[:, 3:4]) * 2.0) ** 2 * an[:, 1:2]
            tx = tb[:, 0:1] * hf - gi
            ty = tb[:, 1:2] * hf - gj
            tw = tb[:, 2:3] * hf
            th = tb[:, 3:4] * hf
            b1x1 = px - pw * 0.5
            b1x2 = px + pw * 0.5
            b1y1 = py - ph * 0.5
            b1y2 = py + ph * 0.5
            b2x1 = tx - tw * 0.5
            b2x2 = tx + tw * 0.5
            b2y1 = ty - th * 0.5
            b2y2 = ty + th * 0.5
            inter = (jnp.maximum(jnp.minimum(b1x2, b2x2) - jnp.maximum(b1x1, b2x1), 0.0)
                     * jnp.maximum(jnp.minimum(b1y2, b2y2) - jnp.maximum(b1y1, b2y1), 0.0))
            w1 = b1x2 - b1x1
            h1 = b1y2 - b1y1 + eps
            w2 = b2x2 - b2x1
            h2 = b2y2 - b2y1 + eps
            union = w1 * h1 + w2 * h2 - inter + eps
            iou = inter / union
            cw = jnp.maximum(b1x2, b2x2) - jnp.minimum(b1x1, b2x1)
            ch = jnp.maximum(b1y2, b2y2) - jnp.minimum(b1y1, b2y1)
            c2 = cw * cw + ch * ch + eps
            rho2 = ((b2x1 + b2x2 - b1x1 - b1x2) ** 2
                    + (b2y1 + b2y2 - b1y1 - b1y2) ** 2) * 0.25
            dat = _atan_pos(w2 / h2) - _atan_pos(w1 / h1)
            v = (4.0 / math.pi ** 2) * dat * dat
            alpha = v / (v - iou + (1.0 + eps))
            ciou = iou - (rho2 / c2 + v * alpha)
            box_loss = box_loss + jnp.sum(1.0 - ciou) * (1.0 / _NM)
            x = pp[:, 5:85]
            cls_sp = jnp.sum(_softplus(x))
            cls_idx = lax.broadcasted_iota(jnp.int32, (_NM, _NC), 1)
            tgt = jnp.sum(jnp.where(cls_idx == tcl[lvl][...], x, 0.0))
            cls_loss = cls_loss + (cls_sp - tgt) * (1.0 / (_NM * _NC))
            key_c = kcl[lvl][...]
            key_r = krl[lvl][...]
            eqm = (key_c == key_r).astype(jnp.float32)
            later = (lax.broadcasted_iota(jnp.int32, (_NM, _NM), 1)
                     > lax.broadcasted_iota(jnp.int32, (_NM, _NM), 0)).astype(jnp.float32)
            loser = jnp.max(eqm * later, axis=1, keepdims=True)
            t = jnp.maximum(ciou, 0.0)
            corr = jnp.sum((1.0 - loser) * pp[:, 4:5] * t)
            obj_loss = obj_loss + (acc[lvl] - corr) * (_BAL[lvl] / (_BS * 3 * h * h))
        total = box_loss * 0.05 + obj_loss + cls_loss * 0.5
        out[...] = jnp.reshape(total, (1, 1))


def _tc_loss(pr0, pr1, pr2, pps, kcs, krs, gis, gjs, tbs, ans, tcs):
    full = lambda shape: pl.BlockSpec(shape, lambda i, j: (0,) * len(shape))
    grid = (_BS, 3)
    in_specs = [
        pl.BlockSpec((1, 1, 20, 20), lambda i, j: (i, 4 + 85 * j, 0, 0)),
        pl.BlockSpec((1, 1, 40, 40), lambda i, j: (i, 4 + 85 * j, 0, 0)),
        pl.BlockSpec((1, 1, 80, 80), lambda i, j: (i, 4 + 85 * j, 0, 0)),
    ]
    args = [pr0, pr1, pr2]
    for group in (pps, kcs, krs):
        for arr in group:
            in_specs.append(full(arr.shape))
            args.append(arr)
    for gi, gj in zip(gis, gjs):
        in_specs.append(full(gi.shape))
        args.append(gi)
        in_specs.append(full(gj.shape))
        args.append(gj)
    for group in (tbs, ans, tcs):
        for arr in group:
            in_specs.append(full(arr.shape))
            args.append(arr)
    return pl.pallas_call(
        _tc_loss_body,
        grid=grid,
        in_specs=in_specs,
        out_specs=pl.BlockSpec((1, 1), lambda i, j: (0, 0)),
        out_shape=jax.ShapeDtypeStruct((1, 1), jnp.float32),
        scratch_shapes=[pltpu.SMEM((3,), jnp.float32)],
    )(*args)


def kernel(pred0, pred1, pred2, tbox0, tbox1, tbox2, anch0, anch1, anch2,
           b0, a0, gj0, gi0, tcls0, b1, a1, gj1, gi1, tcls1,
           b2, a2, gj2, gi2, tcls2):
    f32 = jnp.float32
    i32 = jnp.int32
    preds = (pred0, pred1, pred2)
    flats = tuple(p.reshape(-1) for p in preds)
    idxs = jnp.stack([b0, a0, gj0, gi0, b1, a1, gj1, gi1,
                      b2, a2, gj2, gi2]).astype(i32).reshape(12 * _NM)
    out0, out1, out2, key0, key1, key2 = _sc_gather(flats[0], flats[1], flats[2], idxs)
    pps, kcs, krs = [], [], []
    for out, key in ((out0, key0), (out1, key1), (out2, key2)):
        pp = out[:, :_NPT * _SLOT]
        pps.append(pp.reshape(_NTILES * _NPT, _SLOT)[:_NM, :85])
        kf = key.reshape(_NTILES * 16)
        k300 = kf.reshape(_NTILES, 16)[:, :_NPT].reshape(_NTILES * _NPT)[:_NM]
        kcs.append(k300.reshape(_NM, 1))
        krs.append(k300.reshape(1, _NM))
    gis = tuple(g.astype(f32).reshape(_NM, 1) for g in (gi0, gi1, gi2))
    gjs = tuple(g.astype(f32).reshape(_NM, 1) for g in (gj0, gj1, gj2))
    tbs = (tbox0, tbox1, tbox2)
    ans = (anch0, anch1, anch2)
    tcs = tuple(t.astype(i32).reshape(_NM, 1) for t in (tcls0, tcls1, tcls2))
    loss = _tc_loss(preds[0], preds[1], preds[2], pps, kcs, krs, gis, gjs, tbs, ans, tcs)
    return loss.reshape(1)


# speedup vs baseline: 3.0696x; 2.0369x over previous
import functools
import math

import jax
import jax.numpy as jnp
from jax import lax
from jax.experimental import pallas as pl
from jax.experimental.pallas import tpu as pltpu
from jax.experimental.pallas import tpu_sc as plsc

_HW = (20, 40, 80)
_BS = 16
_NC = 80
_NM = 300
_NTILES = 32
_NPT = 10
_SLOT = 96
_TSLOTS = 1024
_BAL = (0.4, 1.0, 4.0)
_RB = (200, 800, 3200)


def _sc_gather(f0, f1, f2, idxs):
    mesh = plsc.VectorSubcoreMesh(core_axis_name="c", subcore_axis_name="s")
    f32 = jnp.float32
    i32 = jnp.int32

    @functools.partial(
        pl.kernel,
        mesh=mesh,
        out_type=[jax.ShapeDtypeStruct((_NTILES, _TSLOTS), f32) for _ in range(3)]
        + [jax.ShapeDtypeStruct((_NTILES, 16), f32) for _ in range(3)],
        scratch_types=[
            pltpu.VMEM((12 * _NM,), i32),
            pltpu.VMEM((_TSLOTS // 128, 128), i32),
            pltpu.VMEM((_TSLOTS,), f32),
            pltpu.VMEM((16,), f32),
            pltpu.SemaphoreType.DMA,
        ],
        compiler_params=pltpu.CompilerParams(needs_layout_passes=False,
                                             use_tc_tiling_on_sc=False),
    )
    def body(p0, p1, p2, idxs_hbm, out0, out1, out2, key0, key1, key2,
             idx_v, idxbuf, vals, keybuf, sem):
        cc = lax.axis_index("c")
        ss = lax.axis_index("s")
        wid = ss * 2 + cc
        pltpu.sync_copy(idxs_hbm, idx_v)
        iota = lax.iota(i32, 16)
        n16 = jnp.minimum(wid * _NPT + iota, _NM - 1)
        for lvl, (h, tbl, outr, keyr) in enumerate(
            ((20, p0, out0, key0), (40, p1, out1, key1), (80, p2, out2, key2))):
            o0 = 4 * lvl * _NM
            bv = plsc.load_gather(idx_v, [n16 + o0])
            av = plsc.load_gather(idx_v, [n16 + (o0 + _NM)])
            gjv = plsc.load_gather(idx_v, [n16 + (o0 + 2 * _NM)])
            giv = plsc.load_gather(idx_v, [n16 + (o0 + 3 * _NM)])
            keybuf[...] = (((bv * 3 + av) * h + gjv) * h + giv).astype(f32)
            pltpu.sync_copy(keybuf, keyr.at[wid])
            for nl in range(_NPT):
                nf = jnp.full((16,), jnp.minimum(wid * _NPT + nl, _NM - 1), i32)
                b1 = plsc.load_gather(idx_v, [nf + o0])
                a1 = plsc.load_gather(idx_v, [nf + (o0 + _NM)])
                gj1 = plsc.load_gather(idx_v, [nf + (o0 + 2 * _NM)])
                gi1 = plsc.load_gather(idx_v, [nf + (o0 + 3 * _NM)])
                if lvl == 0:
                    r1 = (gj1 * h + gi1) * _BS + b1
                else:
                    r1 = (b1 * h + gj1) * h + gi1
                base = r1 * 255 + a1 * 85
                for k in range(6):
                    cvec = iota + 16 * k
                    gidx = base + cvec
                    if 16 * (k + 1) > 85:
                        gidx = jnp.where(cvec < 85, gidx, 0)
                    slot = nl * _SLOT + k * 16
                    idxbuf[slot // 128, pl.ds(slot % 128, 16)] = gidx
            for j in range(4):
                slot = _NPT * _SLOT + j * 16
                idxbuf[slot // 128, pl.ds(slot % 128, 16)] = jnp.zeros((16,), i32)
            cps = [
                pltpu.async_copy(tbl.at[idxbuf.at[q]],
                                 vals.at[pl.ds(q * 128, 128)], sem)
                for q in range(_TSLOTS // 128)
            ]
            for cp in cps:
                cp.wait()
            pltpu.sync_copy(vals, outr.at[wid])

    return body(f0, f1, f2, idxs)


def _softplus(x):
    return jnp.maximum(x, 0.0) + jnp.log(1.0 + jnp.exp(-jnp.abs(x)))


def _sigmoid(x):
    return 1.0 / (1.0 + jnp.exp(-x))


_ATAN_C = (9.999999990537e-01, -3.333329671515e-01, 1.999854226698e-01,
           -1.426438979378e-01, 1.095344985227e-01, -8.407879225914e-02,
           5.804045198803e-02, -3.126450654751e-02, 1.096244313837e-02,
           -1.804490179632e-03)


def _atan_pos(x):
    u = jnp.minimum(x, 1.0 / x)
    u2 = u * u
    p = jnp.float32(_ATAN_C[-1])
    for coef in _ATAN_C[-2::-1]:
        p = p * u2 + coef
    r = u * p
    return jnp.where(x > 1.0, jnp.float32(math.pi / 2) - r, r)


def _tc_loss_body(t0, t1, t2,
                  pp0, pp1, pp2, kc0, kc1, kc2, kr0, kr1, kr2,
                  gi0, gj0, gi1, gj1, gi2, gj2,
                  tb0, tb1, tb2, an0, an1, an2, tc0, tc1, tc2,
                  out, acc):
    i = pl.program_id(0)

    @pl.when(i == 0)
    def _init():
        for lvl in range(3):
            acc[lvl] = 0.0

    for lvl, t in enumerate((t0, t1, t2)):
        blk = t[...]
        s = (jnp.sum(_softplus(blk[:, 4:5])) + jnp.sum(_softplus(blk[:, 89:90]))
             + jnp.sum(_softplus(blk[:, 174:175])))
        acc[lvl] = acc[lvl] + s

    @pl.when(i == _NTILES - 1)
    def _final():
        eps = 1e-7
        box_loss = jnp.float32(0.0)
        cls_loss = jnp.float32(0.0)
        obj_loss = jnp.float32(0.0)
        ppl = (pp0, pp1, pp2)
        kcl = (kc0, kc1, kc2)
        krl = (kr0, kr1, kr2)
        gil = (gi0, gi1, gi2)
        gjl = (gj0, gj1, gj2)
        tbl = (tb0, tb1, tb2)
        anl = (an0, an1, an2)
        tcl = (tc0, tc1, tc2)
        for lvl, h in enumerate(_HW):
            pp = ppl[lvl][...]
            gi = gil[lvl][...]
            gj = gjl[lvl][...]
            tb = tbl[lvl][...]
            an = anl[lvl][...]
            hf = jnp.float32(h)
            px = _sigmoid(pp[:, 0:1]) * 2.0 - 0.5
            py = _sigmoid(pp[:, 1:2]) * 2.0 - 0.5
            pw = (_sigmoid(pp[:, 2:3]) * 2.0) ** 2 * an[:, 0:1]
            ph = (_sigmoid(pp[:, 3:4]) * 2.0) ** 2 * an[:, 1:2]
            tx = tb[:, 0:1] * hf - gi
            ty = tb[:, 1:2] * hf - gj
            tw = tb[:, 2:3] * hf
            th = tb[:, 3:4] * hf
            b1x1 = px - pw * 0.5
            b1x2 = px + pw * 0.5
            b1y1 = py - ph * 0.5
            b1y2 = py + ph * 0.5
            b2x1 = tx - tw * 0.5
            b2x2 = tx + tw * 0.5
            b2y1 = ty - th * 0.5
            b2y2 = ty + th * 0.5
            inter = (jnp.maximum(jnp.minimum(b1x2, b2x2) - jnp.maximum(b1x1, b2x1), 0.0)
                     * jnp.maximum(jnp.minimum(b1y2, b2y2) - jnp.maximum(b1y1, b2y1), 0.0))
            w1 = b1x2 - b1x1
            h1 = b1y2 - b1y1 + eps
            w2 = b2x2 - b2x1
            h2 = b2y2 - b2y1 + eps
            union = w1 * h1 + w2 * h2 - inter + eps
            iou = inter / union
            cw = jnp.maximum(b1x2, b2x2) - jnp.minimum(b1x1, b2x1)
            ch = jnp.maximum(b1y2, b2y2) - jnp.minimum(b1y1, b2y1)
            c2 = cw * cw + ch * ch + eps
            rho2 = ((b2x1 + b2x2 - b1x1 - b1x2) ** 2
                    + (b2y1 + b2y2 - b1y1 - b1y2) ** 2) * 0.25
            dat = _atan_pos(w2 / h2) - _atan_pos(w1 / h1)
            v = (4.0 / math.pi ** 2) * dat * dat
            alpha = v / (v - iou + (1.0 + eps))
            ciou = iou - (rho2 / c2 + v * alpha)
            box_loss = box_loss + jnp.sum(1.0 - ciou) * (1.0 / _NM)
            x = pp[:, 5:85]
            cls_sp = jnp.sum(_softplus(x))
            cls_idx = lax.broadcasted_iota(jnp.int32, (_NM, _NC), 1)
            tgt = jnp.sum(jnp.where(cls_idx == tcl[lvl][...], x, 0.0))
            cls_loss = cls_loss + (cls_sp - tgt) * (1.0 / (_NM * _NC))
            key_c = kcl[lvl][...]
            key_r = krl[lvl][...]
            eqm = (key_c == key_r).astype(jnp.float32)
            later = (lax.broadcasted_iota(jnp.int32, (_NM, _NM), 1)
                     > lax.broadcasted_iota(jnp.int32, (_NM, _NM), 0)).astype(jnp.float32)
            loser = jnp.max(eqm * later, axis=1, keepdims=True)
            t = jnp.maximum(ciou, 0.0)
            corr = jnp.sum((1.0 - loser) * pp[:, 4:5] * t)
            obj_loss = obj_loss + (acc[lvl] - corr) * (_BAL[lvl] / (_BS * 3 * h * h))
        total = box_loss * 0.05 + obj_loss + cls_loss * 0.5
        out[...] = jnp.reshape(total, (1, 1))


def _tc_loss(ts, pps, kcs, krs, gis, gjs, tbs, ans, tcs):
    full = lambda shape: pl.BlockSpec(shape, lambda i: (0,) * len(shape))
    in_specs = [
        pl.BlockSpec((_RB[0], 255), lambda i: (i, 0)),
        pl.BlockSpec((_RB[1], 255), lambda i: (i, 0)),
        pl.BlockSpec((_RB[2], 255), lambda i: (i, 0)),
    ]
    args = list(ts)
    for group in (pps, kcs, krs):
        for arr in group:
            in_specs.append(full(arr.shape))
            args.append(arr)
    for gi, gj in zip(gis, gjs):
        in_specs.append(full(gi.shape))
        args.append(gi)
        in_specs.append(full(gj.shape))
        args.append(gj)
    for group in (tbs, ans, tcs):
        for arr in group:
            in_specs.append(full(arr.shape))
            args.append(arr)
    return pl.pallas_call(
        _tc_loss_body,
        grid=(_NTILES,),
        in_specs=in_specs,
        out_specs=pl.BlockSpec((1, 1), lambda i: (0, 0)),
        out_shape=jax.ShapeDtypeStruct((1, 1), jnp.float32),
        scratch_shapes=[pltpu.SMEM((3,), jnp.float32)],
    )(*args)


def kernel(pred0, pred1, pred2, tbox0, tbox1, tbox2, anch0, anch1, anch2,
           b0, a0, gj0, gi0, tcls0, b1, a1, gj1, gi1, tcls1,
           b2, a2, gj2, gi2, tcls2):
    f32 = jnp.float32
    i32 = jnp.int32
    t0 = jnp.transpose(pred0, (2, 3, 0, 1)).reshape(_HW[0] * _HW[0] * _BS, 255)
    t1 = jnp.transpose(pred1, (0, 2, 3, 1)).reshape(_BS * _HW[1] * _HW[1], 255)
    t2 = jnp.transpose(pred2, (0, 2, 3, 1)).reshape(_BS * _HW[2] * _HW[2], 255)
    f0, f1, f2 = t0.reshape(-1), t1.reshape(-1), t2.reshape(-1)
    idxs = jnp.stack([b0, a0, gj0, gi0, b1, a1, gj1, gi1,
                      b2, a2, gj2, gi2]).astype(i32).reshape(12 * _NM)
    out0, out1, out2, key0, key1, key2 = _sc_gather(f0, f1, f2, idxs)
    pps, kcs, krs = [], [], []
    for out, key in ((out0, key0), (out1, key1), (out2, key2)):
        pp = out[:, :_NPT * _SLOT]
        pps.append(pp.reshape(_NTILES * _NPT, _SLOT)[:_NM, :85])
        k300 = key[:, :_NPT].reshape(_NTILES * _NPT)[:_NM]
        kcs.append(k300.reshape(_NM, 1))
        krs.append(k300.reshape(1, _NM))
    gis = tuple(g.astype(f32).reshape(_NM, 1) for g in (gi0, gi1, gi2))
    gjs = tuple(g.astype(f32).reshape(_NM, 1) for g in (gj0, gj1, gj2))
    tbs = (tbox0, tbox1, tbox2)
    ans = (anch0, anch1, anch2)
    tcs = tuple(t.astype(i32).reshape(_NM, 1) for t in (tcls0, tcls1, tcls2))
    loss = _tc_loss((t0, t1, t2), pps, kcs, krs, gis, gjs, tbs, ans, tcs)
    return loss.reshape(1)


# speedup vs baseline: 3.4758x; 1.1323x over previous
import functools
import math

import jax
import jax.numpy as jnp
from jax import lax
from jax.experimental import pallas as pl
from jax.experimental.pallas import tpu as pltpu
from jax.experimental.pallas import tpu_sc as plsc

_HW = (20, 40, 80)
_BS = 16
_NC = 80
_NM = 300
_NTILES = 32
_NPT = 10
_SLOT = 96
_TSLOTS = 1024
_BAL = (0.4, 1.0, 4.0)
_RB = (200, 800, 3200)


def _sc_gather(f0, f1, f2, idxs):
    mesh = plsc.VectorSubcoreMesh(core_axis_name="c", subcore_axis_name="s")
    f32 = jnp.float32
    i32 = jnp.int32

    @functools.partial(
        pl.kernel,
        mesh=mesh,
        out_type=[jax.ShapeDtypeStruct((_NTILES, _TSLOTS), f32) for _ in range(3)]
        + [jax.ShapeDtypeStruct((_NTILES, 48), f32)],
        scratch_types=[
            pltpu.VMEM((12 * _NM,), i32),
            pltpu.VMEM((3 * _TSLOTS // 128, 128), i32),
            pltpu.VMEM((3, _TSLOTS), f32),
            pltpu.VMEM((48,), f32),
            pltpu.SemaphoreType.DMA,
            pltpu.SemaphoreType.DMA,
        ],
        compiler_params=pltpu.CompilerParams(needs_layout_passes=False,
                                             use_tc_tiling_on_sc=False),
    )
    def body(p0, p1, p2, idxs_hbm, out0, out1, out2, keyo,
             idx_v, idxbuf, vals, keybuf, sem, osem):
        cc = lax.axis_index("c")
        ss = lax.axis_index("s")
        wid = ss * 2 + cc
        pltpu.sync_copy(idxs_hbm, idx_v)
        iota = lax.iota(i32, 16)
        n16 = jnp.minimum(wid * _NPT + iota, _NM - 1)
        cps = []
        for lvl, (h, tbl, outr) in enumerate(
            ((20, p0, out0), (40, p1, out1), (80, p2, out2))):
            o0 = 4 * lvl * _NM
            bv = plsc.load_gather(idx_v, [n16 + o0])
            av = plsc.load_gather(idx_v, [n16 + (o0 + _NM)])
            gjv = plsc.load_gather(idx_v, [n16 + (o0 + 2 * _NM)])
            giv = plsc.load_gather(idx_v, [n16 + (o0 + 3 * _NM)])
            keybuf[pl.ds(16 * lvl, 16)] = (
                ((bv * 3 + av) * h + gjv) * h + giv).astype(f32)
            for nl in range(_NPT):
                nf = jnp.full((16,), jnp.minimum(wid * _NPT + nl, _NM - 1), i32)
                b1 = plsc.load_gather(idx_v, [nf + o0])
                a1 = plsc.load_gather(idx_v, [nf + (o0 + _NM)])
                gj1 = plsc.load_gather(idx_v, [nf + (o0 + 2 * _NM)])
                gi1 = plsc.load_gather(idx_v, [nf + (o0 + 3 * _NM)])
                if lvl == 0:
                    r1 = (gj1 * h + gi1) * _BS + b1
                else:
                    r1 = (b1 * h + gj1) * h + gi1
                base = r1 * 255 + a1 * 85
                for k in range(6):
                    cvec = iota + 16 * k
                    gidx = base + cvec
                    if 16 * (k + 1) > 85:
                        gidx = jnp.where(cvec < 85, gidx, 0)
                    slot = nl * _SLOT + k * 16
                    row = lvl * (_TSLOTS // 128) + slot // 128
                    idxbuf[row, pl.ds(slot % 128, 16)] = gidx
            for j in range(4):
                slot = _NPT * _SLOT + j * 16
                row = lvl * (_TSLOTS // 128) + slot // 128
                idxbuf[row, pl.ds(slot % 128, 16)] = jnp.zeros((16,), i32)
            for q in range(_TSLOTS // 128):
                cps.append(pltpu.async_copy(
                    tbl.at[idxbuf.at[lvl * (_TSLOTS // 128) + q]],
                    vals.at[lvl, pl.ds(q * 128, 128)], sem))
        pltpu.sync_copy(keybuf, keyo.at[wid])
        for cp in cps:
            cp.wait()
        ocps = [pltpu.async_copy(vals.at[lvl], outr.at[wid], osem)
                for lvl, outr in ((0, out0), (1, out1), (2, out2))]
        for cp in ocps:
            cp.wait()

    return body(f0, f1, f2, idxs)


def _softplus(x):
    return jnp.maximum(x, 0.0) + jnp.log(1.0 + jnp.exp(-jnp.abs(x)))


def _sigmoid(x):
    return 1.0 / (1.0 + jnp.exp(-x))


_ATAN_C = (9.999999990537e-01, -3.333329671515e-01, 1.999854226698e-01,
           -1.426438979378e-01, 1.095344985227e-01, -8.407879225914e-02,
           5.804045198803e-02, -3.126450654751e-02, 1.096244313837e-02,
           -1.804490179632e-03)


def _atan_pos(x):
    u = jnp.minimum(x, 1.0 / x)
    u2 = u * u
    p = jnp.float32(_ATAN_C[-1])
    for coef in _ATAN_C[-2::-1]:
        p = p * u2 + coef
    r = u * p
    return jnp.where(x > 1.0, jnp.float32(math.pi / 2) - r, r)


def _tc_obj_body(t0, t1, t2, acc):
    i = pl.program_id(0)

    @pl.when(i == 0)
    def _init():
        for lvl in range(3):
            acc[lvl] = 0.0

    for lvl, t in enumerate((t0, t1, t2)):
        blk = t[...]
        lane = lax.broadcasted_iota(jnp.int32, blk.shape, 1)
        mask = jnp.logical_or(jnp.logical_or(lane == 4, lane == 89), lane == 174)
        s = jnp.sum(jnp.where(mask, _softplus(blk), 0.0))
        acc[lvl] = acc[lvl] + s


def _tc_obj(ts):
    return pl.pallas_call(
        _tc_obj_body,
        grid=(_NTILES,),
        in_specs=[
            pl.BlockSpec((_RB[0], 255), lambda i: (i, 0)),
            pl.BlockSpec((_RB[1], 255), lambda i: (i, 0)),
            pl.BlockSpec((_RB[2], 255), lambda i: (i, 0)),
        ],
        out_specs=pl.BlockSpec(memory_space=pltpu.SMEM),
        out_shape=jax.ShapeDtypeStruct((3,), jnp.float32),
    )(*ts)


def _tc_final_body(acc,
                  pp0, pp1, pp2, kc0, kc1, kc2, kr0, kr1, kr2,
                  gi0, gj0, gi1, gj1, gi2, gj2,
                  tb0, tb1, tb2, an0, an1, an2, tc0, tc1, tc2,
                  out):
    if True:
        eps = 1e-7
        box_loss = jnp.float32(0.0)
        cls_loss = jnp.float32(0.0)
        obj_loss = jnp.float32(0.0)
        ppl = (pp0, pp1, pp2)
        kcl = (kc0, kc1, kc2)
        krl = (kr0, kr1, kr2)
        gil = (gi0, gi1, gi2)
        gjl = (gj0, gj1, gj2)
        tbl = (tb0, tb1, tb2)
        anl = (an0, an1, an2)
        tcl = (tc0, tc1, tc2)
        for lvl, h in enumerate(_HW):
            pp = ppl[lvl][...]
            gi = gil[lvl][...]
            gj = gjl[lvl][...]
            tb = tbl[lvl][...]
            an = anl[lvl][...]
            hf = jnp.float32(h)
            px = _sigmoid(pp[:, 0:1]) * 2.0 - 0.5
            py = _sigmoid(pp[:, 1:2]) * 2.0 - 0.5
            pw = (_sigmoid(pp[:, 2:3]) * 2.0) ** 2 * an[:, 0:1]
            ph = (_sigmoid(pp[:, 3:4]) * 2.0) ** 2 * an[:, 1:2]
            tx = tb[:, 0:1] * hf - gi
            ty = tb[:, 1:2] * hf - gj
            tw = tb[:, 2:3] * hf
            th = tb[:, 3:4] * hf
            b1x1 = px - pw * 0.5
            b1x2 = px + pw * 0.5
            b1y1 = py - ph * 0.5
            b1y2 = py + ph * 0.5
            b2x1 = tx - tw * 0.5
            b2x2 = tx + tw * 0.5
            b2y1 = ty - th * 0.5
            b2y2 = ty + th * 0.5
            inter = (jnp.maximum(jnp.minimum(b1x2, b2x2) - jnp.maximum(b1x1, b2x1), 0.0)
                     * jnp.maximum(jnp.minimum(b1y2, b2y2) - jnp.maximum(b1y1, b2y1), 0.0))
            w1 = b1x2 - b1x1
            h1 = b1y2 - b1y1 + eps
            w2 = b2x2 - b2x1
            h2 = b2y2 - b2y1 + eps
            union = w1 * h1 + w2 * h2 - inter + eps
            iou = inter / union
            cw = jnp.maximum(b1x2, b2x2) - jnp.minimum(b1x1, b2x1)
            ch = jnp.maximum(b1y2, b2y2) - jnp.minimum(b1y1, b2y1)
            c2 = cw * cw + ch * ch + eps
            rho2 = ((b2x1 + b2x2 - b1x1 - b1x2) ** 2
                    + (b2y1 + b2y2 - b1y1 - b1y2) ** 2) * 0.25
            dat = _atan_pos(w2 / h2) - _atan_pos(w1 / h1)
            v = (4.0 / math.pi ** 2) * dat * dat
            alpha = v / (v - iou + (1.0 + eps))
            ciou = iou - (rho2 / c2 + v * alpha)
            box_loss = box_loss + jnp.sum(1.0 - ciou) * (1.0 / _NM)
            x = pp[:, 5:85]
            cls_sp = jnp.sum(_softplus(x))
            cls_idx = lax.broadcasted_iota(jnp.int32, (_NM, _NC), 1)
            tgt = jnp.sum(jnp.where(cls_idx == tcl[lvl][...], x, 0.0))
            cls_loss = cls_loss + (cls_sp - tgt) * (1.0 / (_NM * _NC))
            key_c = kcl[lvl][...]
            key_r = krl[lvl][...]
            eqm = (key_c == key_r).astype(jnp.float32)
            later = (lax.broadcasted_iota(jnp.int32, (_NM, _NM), 1)
                     > lax.broadcasted_iota(jnp.int32, (_NM, _NM), 0)).astype(jnp.float32)
            loser = jnp.max(eqm * later, axis=1, keepdims=True)
            t = jnp.maximum(ciou, 0.0)
            corr = jnp.sum((1.0 - loser) * pp[:, 4:5] * t)
            obj_loss = obj_loss + (acc[lvl] - corr) * (_BAL[lvl] / (_BS * 3 * h * h))
        total = box_loss * 0.05 + obj_loss + cls_loss * 0.5
        out[...] = jnp.reshape(total, (1, 1))


def _tc_final(acc3, pps, kcs, krs, gis, gjs, tbs, ans, tcs):
    full = lambda shape: pl.BlockSpec(shape, lambda: (0,) * len(shape))
    in_specs = [pl.BlockSpec(memory_space=pltpu.SMEM)]
    args = [acc3]
    for group in (pps, kcs, krs):
        for arr in group:
            in_specs.append(full(arr.shape))
            args.append(arr)
    for gi, gj in zip(gis, gjs):
        in_specs.append(full(gi.shape))
        args.append(gi)
        in_specs.append(full(gj.shape))
        args.append(gj)
    for group in (tbs, ans, tcs):
        for arr in group:
            in_specs.append(full(arr.shape))
            args.append(arr)
    return pl.pallas_call(
        _tc_final_body,
        in_specs=in_specs,
        out_specs=pl.BlockSpec((1, 1), lambda: (0, 0)),
        out_shape=jax.ShapeDtypeStruct((1, 1), jnp.float32),
    )(*args)


def kernel(pred0, pred1, pred2, tbox0, tbox1, tbox2, anch0, anch1, anch2,
           b0, a0, gj0, gi0, tcls0, b1, a1, gj1, gi1, tcls1,
           b2, a2, gj2, gi2, tcls2):
    f32 = jnp.float32
    i32 = jnp.int32
    t0 = jnp.transpose(pred0, (2, 3, 0, 1)).reshape(_HW[0] * _HW[0] * _BS, 255)
    t1 = jnp.transpose(pred1, (0, 2, 3, 1)).reshape(_BS * _HW[1] * _HW[1], 255)
    t2 = jnp.transpose(pred2, (0, 2, 3, 1)).reshape(_BS * _HW[2] * _HW[2], 255)
    f0, f1, f2 = t0.reshape(-1), t1.reshape(-1), t2.reshape(-1)
    idxs = jnp.stack([b0, a0, gj0, gi0, b1, a1, gj1, gi1,
                      b2, a2, gj2, gi2]).astype(i32).reshape(12 * _NM)
    out0, out1, out2, keyo = _sc_gather(f0, f1, f2, idxs)
    pps, kcs, krs = [], [], []
    for lvl, out in enumerate((out0, out1, out2)):
        pp = out[:, :_NPT * _SLOT]
        pps.append(pp.reshape(_NTILES * _NPT, _SLOT)[:_NM, :85])
        k300 = keyo[:, 16 * lvl:16 * lvl + _NPT].reshape(_NTILES * _NPT)[:_NM]
        kcs.append(k300.reshape(_NM, 1))
        krs.append(k300.reshape(1, _NM))
    gis = tuple(g.astype(f32).reshape(_NM, 1) for g in (gi0, gi1, gi2))
    gjs = tuple(g.astype(f32).reshape(_NM, 1) for g in (gj0, gj1, gj2))
    tbs = (tbox0, tbox1, tbox2)
    ans = (anch0, anch1, anch2)
    tcs = tuple(t.astype(i32).reshape(_NM, 1) for t in (tcls0, tcls1, tcls2))
    acc3 = _tc_obj((t0, t1, t2))
    loss = _tc_final(acc3, pps, kcs, krs, gis, gjs, tbs, ans, tcs)
    return loss.reshape(1)


# speedup vs baseline: 4.9563x; 1.4259x over previous
import functools
import math

import jax
import jax.numpy as jnp
from jax import lax
from jax.experimental import pallas as pl
from jax.experimental.pallas import tpu as pltpu
from jax.experimental.pallas import tpu_sc as plsc

_HW = (20, 40, 80)
_BS = 16
_NC = 80
_NM = 300
_NTILES = 32
_NPT = 10
_SLOT = 96
_TSLOTS = 1024
_BAL = (0.4, 1.0, 4.0)
_RB = (200, 800, 3200)


def _sc_gather(f0, f1, f2, idxs):
    mesh = plsc.VectorSubcoreMesh(core_axis_name="c", subcore_axis_name="s")
    f32 = jnp.float32
    i32 = jnp.int32

    @functools.partial(
        pl.kernel,
        mesh=mesh,
        out_type=[jax.ShapeDtypeStruct((_NTILES, _TSLOTS), f32) for _ in range(3)]
        + [jax.ShapeDtypeStruct((_NTILES, 48), f32)],
        scratch_types=[
            pltpu.VMEM((12 * _NM,), i32),
            pltpu.VMEM((3 * _TSLOTS // 128, 128), i32),
            pltpu.VMEM((3, _TSLOTS), f32),
            pltpu.VMEM((48,), f32),
            pltpu.SemaphoreType.DMA,
            pltpu.SemaphoreType.DMA,
        ],
        compiler_params=pltpu.CompilerParams(needs_layout_passes=False,
                                             use_tc_tiling_on_sc=False),
    )
    def body(p0, p1, p2, idxs_hbm, out0, out1, out2, keyo,
             idx_v, idxbuf, vals, keybuf, sem, osem):
        cc = lax.axis_index("c")
        ss = lax.axis_index("s")
        wid = ss * 2 + cc
        pltpu.sync_copy(idxs_hbm, idx_v)
        iota = lax.iota(i32, 16)
        n16 = jnp.minimum(wid * _NPT + iota, _NM - 1)
        cps = []
        for lvl, (h, tbl, outr) in enumerate(
            ((20, p0, out0), (40, p1, out1), (80, p2, out2))):
            o0 = 4 * lvl * _NM
            bv = plsc.load_gather(idx_v, [n16 + o0])
            av = plsc.load_gather(idx_v, [n16 + (o0 + _NM)])
            gjv = plsc.load_gather(idx_v, [n16 + (o0 + 2 * _NM)])
            giv = plsc.load_gather(idx_v, [n16 + (o0 + 3 * _NM)])
            keybuf[pl.ds(16 * lvl, 16)] = (
                ((bv * 3 + av) * h + gjv) * h + giv).astype(f32)
            for nl in range(_NPT):
                nf = jnp.full((16,), jnp.minimum(wid * _NPT + nl, _NM - 1), i32)
                b1 = plsc.load_gather(idx_v, [nf + o0])
                a1 = plsc.load_gather(idx_v, [nf + (o0 + _NM)])
                gj1 = plsc.load_gather(idx_v, [nf + (o0 + 2 * _NM)])
                gi1 = plsc.load_gather(idx_v, [nf + (o0 + 3 * _NM)])
                if lvl == 0:
                    r1 = (gj1 * h + gi1) * _BS + b1
                else:
                    r1 = (b1 * h + gj1) * h + gi1
                rbase = r1 * 128
                cbase = a1 * 85
                half = _BS * h * h * 128
                for k in range(6):
                    cvec = iota + 16 * k
                    cfull = cbase + cvec
                    gidx = (jnp.where(cfull >= 128, half, 0) + rbase
                            + (cfull & 127))
                    if 16 * (k + 1) > 85:
                        gidx = jnp.where(cvec < 85, gidx, 0)
                    slot = nl * _SLOT + k * 16
                    row = lvl * (_TSLOTS // 128) + slot // 128
                    idxbuf[row, pl.ds(slot % 128, 16)] = gidx
            for j in range(4):
                slot = _NPT * _SLOT + j * 16
                row = lvl * (_TSLOTS // 128) + slot // 128
                idxbuf[row, pl.ds(slot % 128, 16)] = jnp.zeros((16,), i32)
            for q in range(_TSLOTS // 128):
                cps.append(pltpu.async_copy(
                    tbl.at[idxbuf.at[lvl * (_TSLOTS // 128) + q]],
                    vals.at[lvl, pl.ds(q * 128, 128)], sem))
        pltpu.sync_copy(keybuf, keyo.at[wid])
        for cp in cps:
            cp.wait()
        ocps = [pltpu.async_copy(vals.at[lvl], outr.at[wid], osem)
                for lvl, outr in ((0, out0), (1, out1), (2, out2))]
        for cp in ocps:
            cp.wait()

    return body(f0, f1, f2, idxs)


def _softplus(x):
    return jnp.maximum(x, 0.0) + jnp.log(1.0 + jnp.exp(-jnp.abs(x)))


def _sigmoid(x):
    return 1.0 / (1.0 + jnp.exp(-x))


_ATAN_C = (9.999999990537e-01, -3.333329671515e-01, 1.999854226698e-01,
           -1.426438979378e-01, 1.095344985227e-01, -8.407879225914e-02,
           5.804045198803e-02, -3.126450654751e-02, 1.096244313837e-02,
           -1.804490179632e-03)


def _atan_pos(x):
    u = jnp.minimum(x, 1.0 / x)
    u2 = u * u
    p = jnp.float32(_ATAN_C[-1])
    for coef in _ATAN_C[-2::-1]:
        p = p * u2 + coef
    r = u * p
    return jnp.where(x > 1.0, jnp.float32(math.pi / 2) - r, r)


def _tc_obj_body(t0, t1, t2, f0, f1, f2, acc):
    i = pl.program_id(0)

    @pl.when(i == 0)
    def _init():
        for lvl in range(3):
            acc[lvl] = 0.0

    for lvl, (t, f) in enumerate(((t0, f0), (t1, f1), (t2, f2))):
        blk = t[...]
        rb = blk.shape[0]
        lane = lax.broadcasted_iota(jnp.int32, blk.shape, 1)
        mask = jnp.logical_or(jnp.logical_or(lane == 4, lane == 89), lane == 174)
        s = jnp.sum(jnp.where(mask, _softplus(blk), 0.0))
        acc[lvl] = acc[lvl] + s
        pa = blk[:, 0:128].reshape(rb // 8, 8, 128)
        pb = jnp.concatenate(
            [blk[:, 128:255], jnp.zeros((rb, 1), jnp.float32)], axis=1
        ).reshape(rb // 8, 8, 128)
        f[...] = jnp.stack([pa, pb], axis=0)


def _tc_obj(ts):
    rtot = tuple(_BS * h * h for h in _HW)
    return pl.pallas_call(
        _tc_obj_body,
        grid=(_NTILES,),
        in_specs=[
            pl.BlockSpec((_RB[0], 255), lambda i: (i, 0)),
            pl.BlockSpec((_RB[1], 255), lambda i: (i, 0)),
            pl.BlockSpec((_RB[2], 255), lambda i: (i, 0)),
        ],
        out_specs=[
            pl.BlockSpec((2, _RB[0] // 8, 8, 128), lambda i: (0, i, 0, 0)),
            pl.BlockSpec((2, _RB[1] // 8, 8, 128), lambda i: (0, i, 0, 0)),
            pl.BlockSpec((2, _RB[2] // 8, 8, 128), lambda i: (0, i, 0, 0)),
            pl.BlockSpec(memory_space=pltpu.SMEM),
        ],
        out_shape=[
            jax.ShapeDtypeStruct((2, rtot[0] // 8, 8, 128), jnp.float32),
            jax.ShapeDtypeStruct((2, rtot[1] // 8, 8, 128), jnp.float32),
            jax.ShapeDtypeStruct((2, rtot[2] // 8, 8, 128), jnp.float32),
            jax.ShapeDtypeStruct((3,), jnp.float32),
        ],
    )(*ts)


def _tc_final_body(acc,
                  pp0, pp1, pp2, kc0, kc1, kc2, kr0, kr1, kr2,
                  gi0, gj0, gi1, gj1, gi2, gj2,
                  tb0, tb1, tb2, an0, an1, an2, tc0, tc1, tc2,
                  out):
    if True:
        eps = 1e-7
        box_loss = jnp.float32(0.0)
        cls_loss = jnp.float32(0.0)
        obj_loss = jnp.float32(0.0)
        ppl = (pp0, pp1, pp2)
        kcl = (kc0, kc1, kc2)
        krl = (kr0, kr1, kr2)
        gil = (gi0, gi1, gi2)
        gjl = (gj0, gj1, gj2)
        tbl = (tb0, tb1, tb2)
        anl = (an0, an1, an2)
        tcl = (tc0, tc1, tc2)
        for lvl, h in enumerate(_HW):
            pp = ppl[lvl][...]
            gi = gil[lvl][...]
            gj = gjl[lvl][...]
            tb = tbl[lvl][...]
            an = anl[lvl][...]
            hf = jnp.float32(h)
            px = _sigmoid(pp[:, 0:1]) * 2.0 - 0.5
            py = _sigmoid(pp[:, 1:2]) * 2.0 - 0.5
            pw = (_sigmoid(pp[:, 2:3]) * 2.0) ** 2 * an[:, 0:1]
            ph = (_sigmoid(pp[:, 3:4]) * 2.0) ** 2 * an[:, 1:2]
            tx = tb[:, 0:1] * hf - gi
            ty = tb[:, 1:2] * hf - gj
            tw = tb[:, 2:3] * hf
            th = tb[:, 3:4] * hf
            b1x1 = px - pw * 0.5
            b1x2 = px + pw * 0.5
            b1y1 = py - ph * 0.5
            b1y2 = py + ph * 0.5
            b2x1 = tx - tw * 0.5
            b2x2 = tx + tw * 0.5
            b2y1 = ty - th * 0.5
            b2y2 = ty + th * 0.5
            inter = (jnp.maximum(jnp.minimum(b1x2, b2x2) - jnp.maximum(b1x1, b2x1), 0.0)
                     * jnp.maximum(jnp.minimum(b1y2, b2y2) - jnp.maximum(b1y1, b2y1), 0.0))
            w1 = b1x2 - b1x1
            h1 = b1y2 - b1y1 + eps
            w2 = b2x2 - b2x1
            h2 = b2y2 - b2y1 + eps
            union = w1 * h1 + w2 * h2 - inter + eps
            iou = inter / union
            cw = jnp.maximum(b1x2, b2x2) - jnp.minimum(b1x1, b2x1)
            ch = jnp.maximum(b1y2, b2y2) - jnp.minimum(b1y1, b2y1)
            c2 = cw * cw + ch * ch + eps
            rho2 = ((b2x1 + b2x2 - b1x1 - b1x2) ** 2
                    + (b2y1 + b2y2 - b1y1 - b1y2) ** 2) * 0.25
            dat = _atan_pos(w2 / h2) - _atan_pos(w1 / h1)
            v = (4.0 / math.pi ** 2) * dat * dat
            alpha = v / (v - iou + (1.0 + eps))
            ciou = iou - (rho2 / c2 + v * alpha)
            box_loss = box_loss + jnp.sum(1.0 - ciou) * (1.0 / _NM)
            x = pp[:, 5:85]
            cls_sp = jnp.sum(_softplus(x))
            cls_idx = lax.broadcasted_iota(jnp.int32, (_NM, _NC), 1)
            tgt = jnp.sum(jnp.where(cls_idx == tcl[lvl][...], x, 0.0))
            cls_loss = cls_loss + (cls_sp - tgt) * (1.0 / (_NM * _NC))
            key_c = kcl[lvl][...]
            key_r = krl[lvl][...]
            eqm = (key_c == key_r).astype(jnp.float32)
            later = (lax.broadcasted_iota(jnp.int32, (_NM, _NM), 1)
                     > lax.broadcasted_iota(jnp.int32, (_NM, _NM), 0)).astype(jnp.float32)
            loser = jnp.max(eqm * later, axis=1, keepdims=True)
            t = jnp.maximum(ciou, 0.0)
            corr = jnp.sum((1.0 - loser) * pp[:, 4:5] * t)
            obj_loss = obj_loss + (acc[lvl] - corr) * (_BAL[lvl] / (_BS * 3 * h * h))
        total = box_loss * 0.05 + obj_loss + cls_loss * 0.5
        out[...] = jnp.reshape(total, (1, 1))


def _tc_final(acc3, pps, kcs, krs, gis, gjs, tbs, ans, tcs):
    full = lambda shape: pl.BlockSpec(shape, lambda: (0,) * len(shape))
    in_specs = [pl.BlockSpec(memory_space=pltpu.SMEM)]
    args = [acc3]
    for group in (pps, kcs, krs):
        for arr in group:
            in_specs.append(full(arr.shape))
            args.append(arr)
    for gi, gj in zip(gis, gjs):
        in_specs.append(full(gi.shape))
        args.append(gi)
        in_specs.append(full(gj.shape))
        args.append(gj)
    for group in (tbs, ans, tcs):
        for arr in group:
            in_specs.append(full(arr.shape))
            args.append(arr)
    return pl.pallas_call(
        _tc_final_body,
        in_specs=in_specs,
        out_specs=pl.BlockSpec((1, 1), lambda: (0, 0)),
        out_shape=jax.ShapeDtypeStruct((1, 1), jnp.float32),
    )(*args)


def kernel(pred0, pred1, pred2, tbox0, tbox1, tbox2, anch0, anch1, anch2,
           b0, a0, gj0, gi0, tcls0, b1, a1, gj1, gi1, tcls1,
           b2, a2, gj2, gi2, tcls2):
    f32 = jnp.float32
    i32 = jnp.int32
    t0 = jnp.transpose(pred0, (2, 3, 0, 1)).reshape(_HW[0] * _HW[0] * _BS, 255)
    t1 = jnp.transpose(pred1, (0, 2, 3, 1)).reshape(_BS * _HW[1] * _HW[1], 255)
    t2 = jnp.transpose(pred2, (0, 2, 3, 1)).reshape(_BS * _HW[2] * _HW[2], 255)
    idxs = jnp.stack([b0, a0, gj0, gi0, b1, a1, gj1, gi1,
                      b2, a2, gj2, gi2]).astype(i32).reshape(12 * _NM)
    pk0, pk1, pk2, acc3 = _tc_obj((t0, t1, t2))
    out0, out1, out2, keyo = _sc_gather(pk0.reshape(-1), pk1.reshape(-1),
                                        pk2.reshape(-1), idxs)
    pps, kcs, krs = [], [], []
    for lvl, out in enumerate((out0, out1, out2)):
        pp = out[:, :_NPT * _SLOT]
        pps.append(pp.reshape(_NTILES * _NPT, _SLOT)[:_NM, :85])
        k300 = keyo[:, 16 * lvl:16 * lvl + _NPT].reshape(_NTILES * _NPT)[:_NM]
        kcs.append(k300.reshape(_NM, 1))
        krs.append(k300.reshape(1, _NM))
    gis = tuple(g.astype(f32).reshape(_NM, 1) for g in (gi0, gi1, gi2))
    gjs = tuple(g.astype(f32).reshape(_NM, 1) for g in (gj0, gj1, gj2))
    tbs = (tbox0, tbox1, tbox2)
    ans = (anch0, anch1, anch2)
    tcs = tuple(t.astype(i32).reshape(_NM, 1) for t in (tcls0, tcls1, tcls2))
    loss = _tc_final(acc3, pps, kcs, krs, gis, gjs, tbs, ans, tcs)
    return loss.reshape(1)
